# Initial kernel scaffold; baseline (speedup 1.0000x reference)
import jax
import jax.numpy as jnp
from jax.experimental import pallas as pl


def kernel(edge_index, message_map0, marginal_psi0, beta):
    raise NotImplementedError("write your pallas kernel here")


# speedup vs baseline: 18.9195x; 18.9195x over previous
import functools

import jax
import jax.numpy as jnp
from jax import lax
from jax.experimental import pallas as pl
from jax.experimental.pallas import tpu as pltpu
from jax.experimental.pallas import tpu_sc as plsc

Q = 4
MAX_ITER = 10
MEAN_DEGREE = 16.0

E = 1600000
N = 100000
NP = 100096
NPR = NP // 128
R_EDGE = E * Q // 128
BLK = 2000
EV_S = 50
EV_L = E // EV_S
EV_BL = 3200

NCORES = 2
NSUB = 16
CH = 5000
SCAT_PER_TILE = E // NSUB
SCAT_CHUNKS = SCAT_PER_TILE // CH
GATH_PER_TILE = E // (NCORES * NSUB)
GATH_CHUNKS = GATH_PER_TILE // CH
ZELEM = NP // NSUB
FSLICE = NP // 8

_f32 = jnp.float32
_smem_spec = pl.BlockSpec(memory_space=pltpu.SMEM)


def _lt_body(m_ref, c_ref, o_ref):
    o_ref[...] = jnp.log1p(c_ref[0, 0] * m_ref[...])


def _edge_body_mid(g_ref, a_ref, h_ref, c_ref, o_ref):
    hb = h_ref[...][:, 0:1, None]
    x = g_ref[...] - a_ref[...] - hb
    e = jnp.exp(x)
    w = e / jnp.sum(e, axis=0, keepdims=True)
    o_ref[...] = jnp.log1p(c_ref[0, 0] * w)


def _edge_body_last(g_ref, a_ref, h_ref, c_ref, o_ref):
    hb = h_ref[...][:, 0:1, None]
    x = g_ref[...] - a_ref[...] - hb
    e = jnp.exp(x)
    o_ref[...] = e / jnp.sum(e, axis=0, keepdims=True)


def _valid_mask(shape):
    row = lax.broadcasted_iota(jnp.int32, shape, 1)
    lane = lax.broadcasted_iota(jnp.int32, shape, 2)
    return jnp.logical_or(row < NPR - 1, lane < (128 - (NP - N)))


def _psi_core(f_ref, h_ref, coef_ref):
    hb = h_ref[...][:, 0:1, None]
    x = f_ref[...] - hb
    e = jnp.exp(x)
    psi = e / jnp.sum(e, axis=0, keepdims=True)
    psim = jnp.where(_valid_mask(psi.shape), psi, 0.0)
    t = jnp.sum(psim, axis=(1, 2), keepdims=True)
    hv = coef_ref[0, 0] * jnp.broadcast_to(t[:, :, 0], (Q, 1))
    return psi, jnp.broadcast_to(hv, (Q, 128))


def _psi_body_mid(f_ref, h_ref, coef_ref, ho_ref):
    _, ho_ref[...] = _psi_core(f_ref, h_ref, coef_ref)


def _psi_body_last(f_ref, h_ref, coef_ref, ho_ref, p_ref):
    p_ref[...], ho_ref[...] = _psi_core(f_ref, h_ref, coef_ref)


def _h0_body(p0_ref, coef_ref, ho_ref):
    t = jnp.sum(p0_ref[...], axis=0, keepdims=True)
    lane = lax.broadcasted_iota(jnp.int32, t.shape, 1)
    rows = [
        jnp.sum(jnp.where((lane & 3) == p, t, 0.0), axis=1, keepdims=True)
        for p in range(Q)
    ]
    hv = coef_ref[0, 0] * jnp.concatenate(rows, axis=0)
    ho_ref[...] = jnp.broadcast_to(hv, (Q, 128))


def _edge_call(body, g4, a4, hvec, c11):
    return pl.pallas_call(
        body,
        grid=(EV_L // EV_BL,),
        in_specs=[
            pl.BlockSpec((Q, EV_S, EV_BL), lambda i: (0, 0, i)),
            pl.BlockSpec((Q, EV_S, EV_BL), lambda i: (0, 0, i)),
            pl.BlockSpec((Q, 128), lambda i: (0, 0)),
            _smem_spec,
        ],
        out_specs=pl.BlockSpec((Q, EV_S, EV_BL), lambda i: (0, 0, i)),
        out_shape=jax.ShapeDtypeStruct((Q, EV_S, EV_L), _f32),
    )(g4, a4, hvec, c11)


def _psi_call_mid(fieldp, hvec, coef11):
    return pl.pallas_call(
        _psi_body_mid,
        grid=(1,),
        in_specs=[
            pl.BlockSpec((Q, NPR, 128), lambda i: (0, 0, 0)),
            pl.BlockSpec((Q, 128), lambda i: (0, 0)),
            _smem_spec,
        ],
        out_specs=pl.BlockSpec((Q, 128), lambda i: (0, 0)),
        out_shape=jax.ShapeDtypeStruct((Q, 128), _f32),
    )(fieldp, hvec, coef11)


def _psi_call_last(fieldp, hvec, coef11):
    return pl.pallas_call(
        _psi_body_last,
        grid=(1,),
        in_specs=[
            pl.BlockSpec((Q, NPR, 128), lambda i: (0, 0, 0)),
            pl.BlockSpec((Q, 128), lambda i: (0, 0)),
            _smem_spec,
        ],
        out_specs=[
            pl.BlockSpec((Q, 128), lambda i: (0, 0)),
            pl.BlockSpec((Q, NPR, 128), lambda i: (0, 0, 0)),
        ],
        out_shape=[
            jax.ShapeDtypeStruct((Q, 128), _f32),
            jax.ShapeDtypeStruct((Q, NPR, 128), _f32),
        ],
    )(fieldp, hvec, coef11)


def _sc_step_body(a_hbm, d_hbm, z_hbm, g_hbm, f_hbm,
                  idx_v, v0, v1, v2, v3, fs0, fs1, fs2, fs3):
    cid = lax.axis_index("c")
    sid = lax.axis_index("s")
    wid = cid * NSUB + sid
    planes = ((v0, fs0), (v1, fs1), (v2, fs2), (v3, fs3))

    zb = sid * ZELEM
    for _, fsp in planes:
        pltpu.sync_copy(z_hbm.at[pl.ds(zb, ZELEM)], fsp.at[pl.ds(zb, ZELEM)])
    plsc.subcore_barrier()

    @pl.loop(0, SCAT_CHUNKS)
    def _scat(i):
        base = sid * SCAT_PER_TILE + i * CH
        pltpu.sync_copy(d_hbm.at[pl.ds(base, CH)], idx_v)
        for p, (vp, fsp) in enumerate(planes):
            pltpu.sync_copy(a_hbm.at[p, pl.ds(base, CH)], vp)
            pltpu.sync_copy(vp, fsp.at[idx_v], add=True)

    plsc.subcore_barrier()

    fplane = wid // 8
    fslice = (wid % 8) * FSLICE
    fsp_sel = planes[0][1]
    for p, (_, fsp) in enumerate(planes):
        @pl.when(fplane == p)
        def _fout(fsp=fsp):
            pltpu.sync_copy(fsp.at[pl.ds(fslice, FSLICE)],
                            f_hbm.at[fplane, pl.ds(fslice, FSLICE)])

    @pl.loop(0, GATH_CHUNKS)
    def _gath(i):
        base = cid * (E // NCORES) + sid * GATH_PER_TILE + i * CH
        pltpu.sync_copy(d_hbm.at[pl.ds(base, CH)], idx_v)
        for p, (vp, fsp) in enumerate(planes):
            pltpu.sync_copy(fsp.at[idx_v], vp)
            pltpu.sync_copy(vp, g_hbm.at[p, pl.ds(base, CH)])


@jax.jit
def _sc_step(a4, d1, zeros):
    mesh = plsc.VectorSubcoreMesh(core_axis_name="c", subcore_axis_name="s")
    step = pl.kernel(
        _sc_step_body,
        out_type=[
            jax.ShapeDtypeStruct((Q, E), _f32),
            jax.ShapeDtypeStruct((Q, NP), _f32),
        ],
        mesh=mesh,
        scratch_types=[
            pltpu.VMEM((CH,), jnp.int32),
            pltpu.VMEM((CH,), _f32),
            pltpu.VMEM((CH,), _f32),
            pltpu.VMEM((CH,), _f32),
            pltpu.VMEM((CH,), _f32),
            pltpu.VMEM_SHARED((NP,), _f32),
            pltpu.VMEM_SHARED((NP,), _f32),
            pltpu.VMEM_SHARED((NP,), _f32),
            pltpu.VMEM_SHARED((NP,), _f32),
        ],
        compiler_params=pltpu.CompilerParams(use_tc_tiling_on_sc=False),
    )
    return step(a4, d1, zeros)


def kernel(edge_index, message_map0, marginal_psi0, beta):
    src = edge_index[0].astype(jnp.int32)
    dst = edge_index[1].astype(jnp.int32)
    beta = beta.astype(_f32)
    c11 = jnp.expm1(beta).reshape(1, 1)
    coef11 = (beta * (MEAN_DEGREE / N)).reshape(1, 1)
    zeros = jnp.zeros((NP,), _f32)

    msg0p = message_map0.astype(_f32).T.reshape(R_EDGE, 128)
    psi0_f = marginal_psi0.astype(_f32).reshape(N * Q // 128, 128)

    a = pl.pallas_call(
        _lt_body,
        grid=(R_EDGE // BLK,),
        in_specs=[pl.BlockSpec((BLK, 128), lambda i: (i, 0)), _smem_spec],
        out_specs=pl.BlockSpec((BLK, 128), lambda i: (i, 0)),
        out_shape=jax.ShapeDtypeStruct((R_EDGE, 128), _f32),
    )(msg0p, c11)

    hvec = pl.pallas_call(
        _h0_body,
        grid=(1,),
        in_specs=[pl.BlockSpec((N * Q // 128, 128), lambda i: (0, 0)), _smem_spec],
        out_specs=pl.BlockSpec((Q, 128), lambda i: (0, 0)),
        out_shape=jax.ShapeDtypeStruct((Q, 128), _f32),
    )(psi0_f, coef11)

    w = psip = None
    for k in range(MAX_ITER):
        d1 = dst if k % 2 == 0 else src
        g4, fieldp = _sc_step(a.reshape(Q, E), d1, zeros)
        gv = g4.reshape(Q, EV_S, EV_L)
        av = a.reshape(Q, EV_S, EV_L)
        fv = fieldp.reshape(Q, NPR, 128)
        if k < MAX_ITER - 1:
            a_new = _edge_call(_edge_body_mid, gv, av, hvec, c11)
            hvec = _psi_call_mid(fv, hvec, coef11)
            a = a_new.reshape(R_EDGE, 128)
        else:
            w = _edge_call(_edge_body_last, gv, av, hvec, c11)
            _, psip = _psi_call_last(fv, hvec, coef11)

    msg = w.reshape(Q, E).T
    psi = psip.reshape(Q, NP)[:, :N].T
    return msg, psi


# speedup vs baseline: 22.7357x; 1.2017x over previous
import functools

import jax
import jax.numpy as jnp
from jax import lax
from jax.experimental import pallas as pl
from jax.experimental.pallas import tpu as pltpu
from jax.experimental.pallas import tpu_sc as plsc

Q = 4
MAX_ITER = 10
MEAN_DEGREE = 16.0

E = 1600000
N = 100000
NP = 100096
NPR = NP // 128
R_EDGE = E * Q // 128
BLK = 2000
EV_S = 50
EV_L = E // EV_S
EV_BL = 3200

NCORES = 2
NSUB = 16
CH = 10000
SCAT_PER_TILE = E // NSUB
SCAT_CHUNKS = SCAT_PER_TILE // CH
GATH_PER_TILE = E // (NCORES * NSUB)
GATH_CHUNKS = GATH_PER_TILE // CH
ZELEM = NP // NSUB
FSLICE = NP // 8

_f32 = jnp.float32
_smem_spec = pl.BlockSpec(memory_space=pltpu.SMEM)


def _lt_body(m_ref, c_ref, o_ref):
    o_ref[...] = jnp.log1p(c_ref[0, 0] * m_ref[...])


def _edge_body_mid(g_ref, a_ref, h_ref, c_ref, o_ref):
    hb = h_ref[...][:, 0:1, None]
    x = g_ref[...] - a_ref[...] - hb
    e = jnp.exp(x)
    w = e / jnp.sum(e, axis=0, keepdims=True)
    o_ref[...] = jnp.log1p(c_ref[0, 0] * w)


def _edge_body_last(g_ref, a_ref, h_ref, c_ref, o_ref):
    hb = h_ref[...][:, 0:1, None]
    x = g_ref[...] - a_ref[...] - hb
    e = jnp.exp(x)
    o_ref[...] = e / jnp.sum(e, axis=0, keepdims=True)


def _valid_mask(shape):
    row = lax.broadcasted_iota(jnp.int32, shape, 1)
    lane = lax.broadcasted_iota(jnp.int32, shape, 2)
    return jnp.logical_or(row < NPR - 1, lane < (128 - (NP - N)))


def _psi_core(f_ref, h_ref, coef_ref):
    hb = h_ref[...][:, 0:1, None]
    x = f_ref[...] - hb
    e = jnp.exp(x)
    psi = e / jnp.sum(e, axis=0, keepdims=True)
    psim = jnp.where(_valid_mask(psi.shape), psi, 0.0)
    t = jnp.sum(psim, axis=(1, 2), keepdims=True)
    hv = coef_ref[0, 0] * jnp.broadcast_to(t[:, :, 0], (Q, 1))
    return psi, jnp.broadcast_to(hv, (Q, 128))


def _psi_body_mid(f_ref, h_ref, coef_ref, ho_ref):
    _, ho_ref[...] = _psi_core(f_ref, h_ref, coef_ref)


def _psi_body_last(f_ref, h_ref, coef_ref, ho_ref, p_ref):
    p_ref[...], ho_ref[...] = _psi_core(f_ref, h_ref, coef_ref)


def _h0_body(p0_ref, coef_ref, ho_ref):
    t = jnp.sum(p0_ref[...], axis=0, keepdims=True)
    lane = lax.broadcasted_iota(jnp.int32, t.shape, 1)
    rows = [
        jnp.sum(jnp.where((lane & 3) == p, t, 0.0), axis=1, keepdims=True)
        for p in range(Q)
    ]
    hv = coef_ref[0, 0] * jnp.concatenate(rows, axis=0)
    ho_ref[...] = jnp.broadcast_to(hv, (Q, 128))


def _edge_call(body, g4, a4, hvec, c11):
    return pl.pallas_call(
        body,
        grid=(EV_L // EV_BL,),
        in_specs=[
            pl.BlockSpec((Q, EV_S, EV_BL), lambda i: (0, 0, i)),
            pl.BlockSpec((Q, EV_S, EV_BL), lambda i: (0, 0, i)),
            pl.BlockSpec((Q, 128), lambda i: (0, 0)),
            _smem_spec,
        ],
        out_specs=pl.BlockSpec((Q, EV_S, EV_BL), lambda i: (0, 0, i)),
        out_shape=jax.ShapeDtypeStruct((Q, EV_S, EV_L), _f32),
    )(g4, a4, hvec, c11)


def _psi_call_mid(fieldp, hvec, coef11):
    return pl.pallas_call(
        _psi_body_mid,
        grid=(1,),
        in_specs=[
            pl.BlockSpec((Q, NPR, 128), lambda i: (0, 0, 0)),
            pl.BlockSpec((Q, 128), lambda i: (0, 0)),
            _smem_spec,
        ],
        out_specs=pl.BlockSpec((Q, 128), lambda i: (0, 0)),
        out_shape=jax.ShapeDtypeStruct((Q, 128), _f32),
    )(fieldp, hvec, coef11)


def _psi_call_last(fieldp, hvec, coef11):
    return pl.pallas_call(
        _psi_body_last,
        grid=(1,),
        in_specs=[
            pl.BlockSpec((Q, NPR, 128), lambda i: (0, 0, 0)),
            pl.BlockSpec((Q, 128), lambda i: (0, 0)),
            _smem_spec,
        ],
        out_specs=[
            pl.BlockSpec((Q, 128), lambda i: (0, 0)),
            pl.BlockSpec((Q, NPR, 128), lambda i: (0, 0, 0)),
        ],
        out_shape=[
            jax.ShapeDtypeStruct((Q, 128), _f32),
            jax.ShapeDtypeStruct((Q, NPR, 128), _f32),
        ],
    )(fieldp, hvec, coef11)


def _sc_step_body(a_hbm, d_hbm, z_hbm, g_hbm, f_hbm,
                  idx_v, v0, v1, v2, v3, fs0, fs1, fs2, fs3, lsem, ssem):
    cid = lax.axis_index("c")
    sid = lax.axis_index("s")
    wid = cid * NSUB + sid
    planes = ((v0, fs0), (v1, fs1), (v2, fs2), (v3, fs3))

    zb = sid * ZELEM
    zcps = [
        pltpu.async_copy(z_hbm.at[pl.ds(zb, ZELEM)],
                         fsp.at[pl.ds(zb, ZELEM)], lsem)
        for _, fsp in planes
    ]
    for c in zcps:
        c.wait()
    plsc.subcore_barrier()

    @pl.loop(0, SCAT_CHUNKS)
    def _scat(i):
        base = sid * SCAT_PER_TILE + i * CH
        cps = [pltpu.async_copy(d_hbm.at[pl.ds(base, CH)], idx_v, lsem)]
        cps += [
            pltpu.async_copy(a_hbm.at[p, pl.ds(base, CH)], vp, lsem)
            for p, (vp, _) in enumerate(planes)
        ]
        for c in cps:
            c.wait()
        scs = [
            pltpu.async_copy(vp, fsp.at[idx_v], ssem, add=True)
            for vp, fsp in planes
        ]
        for c in scs:
            c.wait()

    plsc.subcore_barrier()

    fplane = wid // 8
    fslice = (wid % 8) * FSLICE
    fsp_sel = planes[0][1]
    for p, (_, fsp) in enumerate(planes):
        @pl.when(fplane == p)
        def _fout(fsp=fsp):
            pltpu.sync_copy(fsp.at[pl.ds(fslice, FSLICE)],
                            f_hbm.at[fplane, pl.ds(fslice, FSLICE)])

    @pl.loop(0, GATH_CHUNKS)
    def _gath(i):
        base = cid * (E // NCORES) + sid * GATH_PER_TILE + i * CH
        pltpu.async_copy(d_hbm.at[pl.ds(base, CH)], idx_v, lsem).wait()
        gcs = [
            pltpu.async_copy(fsp.at[idx_v], vp, ssem)
            for vp, fsp in planes
        ]
        for c in gcs:
            c.wait()
        wcs = [
            pltpu.async_copy(vp, g_hbm.at[p, pl.ds(base, CH)], lsem)
            for p, (vp, _) in enumerate(planes)
        ]
        for c in wcs:
            c.wait()


@jax.jit
def _sc_step(a4, d1, zeros):
    mesh = plsc.VectorSubcoreMesh(core_axis_name="c", subcore_axis_name="s")
    step = pl.kernel(
        _sc_step_body,
        out_type=[
            jax.ShapeDtypeStruct((Q, E), _f32),
            jax.ShapeDtypeStruct((Q, NP), _f32),
        ],
        mesh=mesh,
        scratch_types=[
            pltpu.VMEM((CH,), jnp.int32),
            pltpu.VMEM((CH,), _f32),
            pltpu.VMEM((CH,), _f32),
            pltpu.VMEM((CH,), _f32),
            pltpu.VMEM((CH,), _f32),
            pltpu.VMEM_SHARED((NP,), _f32),
            pltpu.VMEM_SHARED((NP,), _f32),
            pltpu.VMEM_SHARED((NP,), _f32),
            pltpu.VMEM_SHARED((NP,), _f32),
            pltpu.SemaphoreType.DMA,
            pltpu.SemaphoreType.DMA,
        ],
        compiler_params=pltpu.CompilerParams(use_tc_tiling_on_sc=False),
    )
    return step(a4, d1, zeros)


def kernel(edge_index, message_map0, marginal_psi0, beta):
    src = edge_index[0].astype(jnp.int32)
    dst = edge_index[1].astype(jnp.int32)
    beta = beta.astype(_f32)
    c11 = jnp.expm1(beta).reshape(1, 1)
    coef11 = (beta * (MEAN_DEGREE / N)).reshape(1, 1)
    zeros = jnp.zeros((NP,), _f32)

    msg0p = message_map0.astype(_f32).T.reshape(R_EDGE, 128)
    psi0_f = marginal_psi0.astype(_f32).reshape(N * Q // 128, 128)

    a = pl.pallas_call(
        _lt_body,
        grid=(R_EDGE // BLK,),
        in_specs=[pl.BlockSpec((BLK, 128), lambda i: (i, 0)), _smem_spec],
        out_specs=pl.BlockSpec((BLK, 128), lambda i: (i, 0)),
        out_shape=jax.ShapeDtypeStruct((R_EDGE, 128), _f32),
    )(msg0p, c11)

    hvec = pl.pallas_call(
        _h0_body,
        grid=(1,),
        in_specs=[pl.BlockSpec((N * Q // 128, 128), lambda i: (0, 0)), _smem_spec],
        out_specs=pl.BlockSpec((Q, 128), lambda i: (0, 0)),
        out_shape=jax.ShapeDtypeStruct((Q, 128), _f32),
    )(psi0_f, coef11)

    w = psip = None
    for k in range(MAX_ITER):
        d1 = dst if k % 2 == 0 else src
        g4, fieldp = _sc_step(a.reshape(Q, E), d1, zeros)
        gv = g4.reshape(Q, EV_S, EV_L)
        av = a.reshape(Q, EV_S, EV_L)
        fv = fieldp.reshape(Q, NPR, 128)
        if k < MAX_ITER - 1:
            a_new = _edge_call(_edge_body_mid, gv, av, hvec, c11)
            hvec = _psi_call_mid(fv, hvec, coef11)
            a = a_new.reshape(R_EDGE, 128)
        else:
            w = _edge_call(_edge_body_last, gv, av, hvec, c11)
            _, psip = _psi_call_last(fv, hvec, coef11)

    msg = w.reshape(Q, E).T
    psi = psip.reshape(Q, NP)[:, :N].T
    return msg, psi
